# baseline (speedup 1.0000x reference)
import jax
import jax.numpy as jnp
from jax.experimental import pallas as pl
from jax.experimental.pallas import tpu as pltpu

_INDEX = 50
_ROWS = 200
_FEAT = 64
_BB = 64


def _select_scatter_block(x_ref, src_ref, o_ref):
    o_ref[...] = x_ref[...]
    o_ref[:, _INDEX, :] = src_ref[...]


def kernel(x, src):
    b = x.shape[0]
    out = pl.pallas_call(
        _select_scatter_block,
        grid=(b // _BB,),
        in_specs=[
            pl.BlockSpec((_BB, _ROWS, _FEAT), lambda i: (i, 0, 0)),
            pl.BlockSpec((_BB, _FEAT), lambda i: (i, 0)),
        ],
        out_specs=pl.BlockSpec((_BB, _ROWS, _FEAT), lambda i: (i, 0, 0)),
        out_shape=jax.ShapeDtypeStruct((b, _ROWS, _FEAT), x.dtype),
        compiler_params=pltpu.CompilerParams(
            dimension_semantics=("parallel",),
        ),
    )(x, src)
    return out


# speedup vs baseline: 1.6517x; 1.6517x over previous
import jax
import jax.numpy as jnp
from jax.experimental import pallas as pl
from jax.experimental.pallas import tpu as pltpu

_INDEX = 50
_ROWS = 200
_FEAT = 64
_COLS = _ROWS * _FEAT
_COL0 = _INDEX * _FEAT
_BB = 256
_CB = 3200
_JSTRIP = _COL0 // _CB


def _select_scatter_block(x_ref, src_ref, o_ref):
    o_ref[...] = x_ref[...]

    @pl.when(pl.program_id(1) == _JSTRIP)
    def _():
        o_ref[:, _COL0 - _JSTRIP * _CB:_COL0 - _JSTRIP * _CB + _FEAT] = (
            src_ref[...])


def kernel(x, src):
    b = x.shape[0]
    x2 = x.reshape(b, _COLS)
    out = pl.pallas_call(
        _select_scatter_block,
        grid=(b // _BB, _COLS // _CB),
        in_specs=[
            pl.BlockSpec((_BB, _CB), lambda i, j: (i, j)),
            pl.BlockSpec((_BB, _FEAT), lambda i, j: (i, 0)),
        ],
        out_specs=pl.BlockSpec((_BB, _CB), lambda i, j: (i, j)),
        out_shape=jax.ShapeDtypeStruct((b, _COLS), x.dtype),
        compiler_params=pltpu.CompilerParams(
            dimension_semantics=("parallel", "parallel"),
        ),
    )(x2, src)
    return out.reshape(x.shape)
